# baseline (speedup 1.0000x reference)
import functools

import jax
import jax.numpy as jnp
from jax import lax
from jax.experimental import pallas as pl
from jax.experimental.pallas import tpu as pltpu
from jax.experimental.pallas import tpu_sc as plsc


def _sc_gather3(u, i, j, gamma_u, gamma_i):
    B = u.shape[0]
    K = gamma_u.shape[1]
    info = plsc.get_sparse_core_info()
    nw = info.num_cores * info.num_subcores
    bpw = B // nw
    mesh = plsc.VectorSubcoreMesh(core_axis_name="c", subcore_axis_name="s")

    depth = 32

    @functools.partial(
        pl.kernel,
        out_type=(jax.ShapeDtypeStruct((B, K), jnp.float32),) * 3,
        mesh=mesh,
        scratch_types=[
            pltpu.VMEM((bpw, K), jnp.float32),
            pltpu.VMEM((bpw, K), jnp.float32),
            pltpu.VMEM((bpw, K), jnp.float32),
            pltpu.VMEM((bpw,), jnp.int32),
            pltpu.VMEM((bpw,), jnp.int32),
            pltpu.VMEM((bpw,), jnp.int32),
            pltpu.SemaphoreType.DMA,
        ],
    )
    def body(u_hbm, i_hbm, j_hbm, gu_hbm, gi_hbm, out_u, out_i, out_j,
             rows_u, rows_i, rows_j, su, si, sj, sem):
        wid = lax.axis_index("s") * info.num_cores + lax.axis_index("c")
        base = wid * bpw
        pltpu.sync_copy(u_hbm.at[pl.ds(base, bpw)], su)
        pltpu.sync_copy(i_hbm.at[pl.ds(base, bpw)], si)
        pltpu.sync_copy(j_hbm.at[pl.ds(base, bpw)], sj)

        def _wait3():
            for dst in (rows_u, rows_i, rows_j):
                pltpu.make_async_copy(gu_hbm.at[pl.ds(0, 1)],
                                      dst.at[pl.ds(0, 1)], sem).wait()

        nlane = info.num_lanes

        def issue(g, c):
            vu = su[pl.ds(g * nlane, nlane)]
            vi = si[pl.ds(g * nlane, nlane)]
            vj = sj[pl.ds(g * nlane, nlane)]
            for l in range(nlane):
                r = g * nlane + l
                pltpu.async_copy(gu_hbm.at[pl.ds(vu[l], 1)], rows_u.at[pl.ds(r, 1)], sem)
                pltpu.async_copy(gi_hbm.at[pl.ds(vi[l], 1)], rows_i.at[pl.ds(r, 1)], sem)
                pltpu.async_copy(gi_hbm.at[pl.ds(vj[l], 1)], rows_j.at[pl.ds(r, 1)], sem)

            @pl.when(g >= depth // nlane)
            def _():
                for _ in range(nlane):
                    _wait3()

            return c

        lax.fori_loop(0, bpw // nlane, issue, 0)

        def drain(g, c):
            for _ in range(nlane):
                _wait3()
            return c

        lax.fori_loop(0, min(depth, bpw) // nlane, drain, 0)
        pltpu.sync_copy(rows_u, out_u.at[pl.ds(base, bpw)])
        pltpu.sync_copy(rows_i, out_i.at[pl.ds(base, bpw)])
        pltpu.sync_copy(rows_j, out_j.at[pl.ds(base, bpw)])

    return body(u, i, j, gamma_u, gamma_i)


def _tc_encoder(ukT, W0T, b0, W1T, b1, W2T, b2, block_n=1024):
    NKP, B = ukT.shape
    K = W0T.shape[0]
    nb = B // block_n
    f32 = jnp.float32

    def body(ukT_r, W0T_r, b0_r, W1T_r, b1_r, W2T_r, b2_r, hT_r):
        hT = jnp.dot(W0T_r[...], ukT_r[...], preferred_element_type=f32) + b0_r[...]
        hT = jnp.maximum(jnp.dot(W1T_r[...], hT, preferred_element_type=f32) + b1_r[...], 0.0)
        hT_r[...] = jnp.maximum(
            jnp.dot(W2T_r[...], hT, preferred_element_type=f32) + b2_r[...], 0.0)

    col = lambda b: (0, b)
    rep = lambda b: (0, 0)
    return pl.pallas_call(
        body,
        grid=(nb,),
        in_specs=[
            pl.BlockSpec((NKP, block_n), col),
            pl.BlockSpec((K, NKP), rep),
            pl.BlockSpec((K, 1), rep),
            pl.BlockSpec((K, K), rep),
            pl.BlockSpec((K, 1), rep),
            pl.BlockSpec((K, K), rep),
            pl.BlockSpec((K, 1), rep),
        ],
        out_specs=pl.BlockSpec((K, block_n), col),
        out_shape=jax.ShapeDtypeStruct((K, B), f32),
    )(ukT, W0T, b0, W1T, b1, W2T, b2)


def _tc_combine(hT, tkT, gu, gi, gj, pW0T, pb0, pW1T, pb1, block_n=512):
    K, B = hT.shape
    NKP = pW1T.shape[0]
    nb = B // block_n
    f32 = jnp.float32

    def body(hT_r, tkT_r, gu_r, gi_r, gj_r, pW0T_r, pb0_r, pW1T_r, pb1_r,
             xui_r, xuj_r, kpsT_r, bpr_r, kp_r):
        guT = gu_r[...].T
        giT = gi_r[...].T
        gjT = gj_r[...].T
        luT = hT_r[...] + guT
        xui = jnp.sum(luT * giT, axis=0, keepdims=True)
        xuj = jnp.sum(luT * gjT, axis=0, keepdims=True)
        xui_r[...] = xui
        xuj_r[...] = xuj
        phT = jnp.maximum(
            jnp.dot(pW0T_r[...], luT + giT, preferred_element_type=f32) + pb0_r[...], 0.0)
        logitsT = jnp.dot(pW1T_r[...], phT, preferred_element_type=f32) + pb1_r[...]
        kpsT_r[...] = logitsT
        t = tkT_r[...]
        bce = (jnp.maximum(logitsT, 0.0) - logitsT * t
               + jnp.log1p(jnp.exp(-jnp.abs(logitsT))))
        z = xui - xuj
        logsig = jnp.minimum(z, 0.0) - jnp.log1p(jnp.exp(-jnp.abs(z)))

        @pl.when(pl.program_id(0) == 0)
        def _init():
            bpr_r[...] = jnp.zeros((1, 1), f32)
            kp_r[...] = jnp.zeros((1, 1), f32)

        bpr_r[...] += -jnp.sum(logsig)
        kp_r[...] += jnp.sum(bce)

        @pl.when(pl.program_id(0) == nb - 1)
        def _finish():
            bpr_r[...] = bpr_r[...] * (1.0 / B)
            kp_r[...] = kp_r[...] * (1.0 / (B * NKP))

    col = lambda b: (0, b)
    row = lambda b: (b, 0)
    rep = lambda b: (0, 0)
    in_specs = [
        pl.BlockSpec((K, block_n), col),
        pl.BlockSpec((NKP, block_n), col),
        pl.BlockSpec((block_n, K), row),
        pl.BlockSpec((block_n, K), row),
        pl.BlockSpec((block_n, K), row),
        pl.BlockSpec((K, K), rep),
        pl.BlockSpec((K, 1), rep),
        pl.BlockSpec((NKP, K), rep),
        pl.BlockSpec((NKP, 1), rep),
    ]
    out_specs = [
        pl.BlockSpec((1, block_n), col),
        pl.BlockSpec((1, block_n), col),
        pl.BlockSpec((NKP, block_n), col),
        pl.BlockSpec((1, 1), rep),
        pl.BlockSpec((1, 1), rep),
    ]
    out_shape = [
        jax.ShapeDtypeStruct((1, B), f32),
        jax.ShapeDtypeStruct((1, B), f32),
        jax.ShapeDtypeStruct((NKP, B), f32),
        jax.ShapeDtypeStruct((1, 1), f32),
        jax.ShapeDtypeStruct((1, 1), f32),
    ]
    return pl.pallas_call(
        body,
        grid=(nb,),
        in_specs=in_specs,
        out_specs=out_specs,
        out_shape=out_shape,
    )(hT, tkT, gu, gi, gj, pW0T, pb0, pW1T, pb1)


def kernel(u, i, j, target_kps, user_kps, gamma_i, gamma_u,
           enc_W0, enc_b0, enc_W1, enc_b1, enc_W2, enc_b2,
           proj_W0, proj_b0, proj_W1, proj_b1):
    K = gamma_u.shape[1]
    NKP = user_kps.shape[1]
    gu, gi, gj = _sc_gather3(u.astype(jnp.int32), i.astype(jnp.int32),
                             j.astype(jnp.int32), gamma_u, gamma_i)
    hT = _tc_encoder(user_kps.T, enc_W0.T, enc_b0.reshape(K, 1),
                     enc_W1.T, enc_b1.reshape(K, 1), enc_W2.T, enc_b2.reshape(K, 1))
    xui, xuj, kpsT, bpr, kp = _tc_combine(
        hT, target_kps.T, gu, gi, gj,
        proj_W0.T, proj_b0.reshape(K, 1), proj_W1.T, proj_b1.reshape(NKP, 1))
    return (xui[0], xuj[0], kpsT.T, bpr[0, 0], kp[0, 0])


# speedup vs baseline: 1.0425x; 1.0425x over previous
import functools

import jax
import jax.numpy as jnp
from jax import lax
from jax.experimental import pallas as pl
from jax.experimental.pallas import tpu as pltpu
from jax.experimental.pallas import tpu_sc as plsc


def _sc_gather3(u, i, j, gamma_u, gamma_i):
    B = u.shape[0]
    K = gamma_u.shape[1]
    info = plsc.get_sparse_core_info()
    nw = info.num_cores * info.num_subcores
    bpw = B // nw
    mesh = plsc.VectorSubcoreMesh(core_axis_name="c", subcore_axis_name="s")

    depth = 32

    @functools.partial(
        pl.kernel,
        out_type=(jax.ShapeDtypeStruct((B, K), jnp.float32),) * 3,
        mesh=mesh,
        scratch_types=[
            pltpu.VMEM((bpw, K), jnp.float32),
            pltpu.VMEM((bpw, K), jnp.float32),
            pltpu.VMEM((bpw, K), jnp.float32),
            pltpu.VMEM((bpw,), jnp.int32),
            pltpu.VMEM((bpw,), jnp.int32),
            pltpu.VMEM((bpw,), jnp.int32),
            pltpu.SemaphoreType.DMA,
        ],
    )
    def body(u_hbm, i_hbm, j_hbm, gu_hbm, gi_hbm, out_u, out_i, out_j,
             rows_u, rows_i, rows_j, su, si, sj, sem):
        wid = lax.axis_index("s") * info.num_cores + lax.axis_index("c")
        base = wid * bpw
        pltpu.sync_copy(u_hbm.at[pl.ds(base, bpw)], su)
        pltpu.sync_copy(i_hbm.at[pl.ds(base, bpw)], si)
        pltpu.sync_copy(j_hbm.at[pl.ds(base, bpw)], sj)

        def _wait3():
            for dst in (rows_u, rows_i, rows_j):
                pltpu.make_async_copy(gu_hbm.at[pl.ds(0, 1)],
                                      dst.at[pl.ds(0, 1)], sem).wait()

        nlane = info.num_lanes

        def issue(g, c):
            vu = su[pl.ds(g * nlane, nlane)]
            vi = si[pl.ds(g * nlane, nlane)]
            vj = sj[pl.ds(g * nlane, nlane)]
            for l in range(nlane):
                r = g * nlane + l
                pltpu.async_copy(gu_hbm.at[pl.ds(vu[l], 1)], rows_u.at[pl.ds(r, 1)], sem)
                pltpu.async_copy(gi_hbm.at[pl.ds(vi[l], 1)], rows_i.at[pl.ds(r, 1)], sem)
                pltpu.async_copy(gi_hbm.at[pl.ds(vj[l], 1)], rows_j.at[pl.ds(r, 1)], sem)

            @pl.when(g >= depth // nlane)
            def _():
                for _ in range(nlane):
                    _wait3()

            return c

        lax.fori_loop(0, bpw // nlane, issue, 0)

        def drain(g, c):
            for _ in range(nlane):
                _wait3()
            return c

        lax.fori_loop(0, min(depth, bpw) // nlane, drain, 0)
        pltpu.sync_copy(rows_u, out_u.at[pl.ds(base, bpw)])
        pltpu.sync_copy(rows_i, out_i.at[pl.ds(base, bpw)])
        pltpu.sync_copy(rows_j, out_j.at[pl.ds(base, bpw)])

    return body(u, i, j, gamma_u, gamma_i)


def _tc_fused(ukT, tkT, gu, gi, gj,
              W0T, b0, W1T, b1, W2T, b2,
              pW0T, pb0, pW1T, pb1, block_n=1024):
    NKP, B = ukT.shape
    K = gu.shape[1]
    nb = B // block_n
    f32 = jnp.float32

    def body(ukT_r, tkT_r, gu_r, gi_r, gj_r, W0T_r, b0_r, W1T_r, b1_r,
             W2T_r, b2_r, pW0T_r, pb0_r, pW1T_r, pb1_r,
             xui_r, xuj_r, kpsT_r, bpr_r, kp_r):
        hT = jnp.dot(W0T_r[...], ukT_r[...], preferred_element_type=f32) + b0_r[...]
        hT = jnp.maximum(jnp.dot(W1T_r[...], hT, preferred_element_type=f32) + b1_r[...], 0.0)
        hT = jnp.maximum(jnp.dot(W2T_r[...], hT, preferred_element_type=f32) + b2_r[...], 0.0)
        guT = gu_r[...].T
        giT = gi_r[...].T
        gjT = gj_r[...].T
        luT = hT + guT
        xui = jnp.sum(luT * giT, axis=0, keepdims=True)
        xuj = jnp.sum(luT * gjT, axis=0, keepdims=True)
        xui_r[...] = xui
        xuj_r[...] = xuj
        phT = jnp.maximum(
            jnp.dot(pW0T_r[...], luT + giT, preferred_element_type=f32) + pb0_r[...], 0.0)
        logitsT = jnp.dot(pW1T_r[...], phT, preferred_element_type=f32) + pb1_r[...]
        kpsT_r[...] = logitsT
        t = tkT_r[...]
        bce = (jnp.maximum(logitsT, 0.0) - logitsT * t
               + jnp.log1p(jnp.exp(-jnp.abs(logitsT))))
        z = xui - xuj
        logsig = jnp.minimum(z, 0.0) - jnp.log1p(jnp.exp(-jnp.abs(z)))

        @pl.when(pl.program_id(0) == 0)
        def _init():
            bpr_r[...] = jnp.zeros((1, 1), f32)
            kp_r[...] = jnp.zeros((1, 1), f32)

        bpr_r[...] += -jnp.sum(logsig)
        kp_r[...] += jnp.sum(bce)

        @pl.when(pl.program_id(0) == nb - 1)
        def _finish():
            bpr_r[...] = bpr_r[...] * (1.0 / B)
            kp_r[...] = kp_r[...] * (1.0 / (B * NKP))

    col = lambda b: (0, b)
    row = lambda b: (b, 0)
    rep = lambda b: (0, 0)
    in_specs = [
        pl.BlockSpec((NKP, block_n), col),
        pl.BlockSpec((NKP, block_n), col),
        pl.BlockSpec((block_n, K), row),
        pl.BlockSpec((block_n, K), row),
        pl.BlockSpec((block_n, K), row),
        pl.BlockSpec((K, NKP), rep),
        pl.BlockSpec((K, 1), rep),
        pl.BlockSpec((K, K), rep),
        pl.BlockSpec((K, 1), rep),
        pl.BlockSpec((K, K), rep),
        pl.BlockSpec((K, 1), rep),
        pl.BlockSpec((K, K), rep),
        pl.BlockSpec((K, 1), rep),
        pl.BlockSpec((NKP, K), rep),
        pl.BlockSpec((NKP, 1), rep),
    ]
    out_specs = [
        pl.BlockSpec((1, block_n), col),
        pl.BlockSpec((1, block_n), col),
        pl.BlockSpec((NKP, block_n), col),
        pl.BlockSpec((1, 1), rep),
        pl.BlockSpec((1, 1), rep),
    ]
    out_shape = [
        jax.ShapeDtypeStruct((1, B), f32),
        jax.ShapeDtypeStruct((1, B), f32),
        jax.ShapeDtypeStruct((NKP, B), f32),
        jax.ShapeDtypeStruct((1, 1), f32),
        jax.ShapeDtypeStruct((1, 1), f32),
    ]
    return pl.pallas_call(
        body,
        grid=(nb,),
        in_specs=in_specs,
        out_specs=out_specs,
        out_shape=out_shape,
    )(ukT, tkT, gu, gi, gj,
      W0T, b0, W1T, b1, W2T, b2, pW0T, pb0, pW1T, pb1)


def kernel(u, i, j, target_kps, user_kps, gamma_i, gamma_u,
           enc_W0, enc_b0, enc_W1, enc_b1, enc_W2, enc_b2,
           proj_W0, proj_b0, proj_W1, proj_b1):
    K = gamma_u.shape[1]
    NKP = user_kps.shape[1]
    gu, gi, gj = _sc_gather3(u.astype(jnp.int32), i.astype(jnp.int32),
                             j.astype(jnp.int32), gamma_u, gamma_i)
    xui, xuj, kpsT, bpr, kp = _tc_fused(
        user_kps.T, target_kps.T, gu, gi, gj,
        enc_W0.T, enc_b0.reshape(K, 1), enc_W1.T, enc_b1.reshape(K, 1),
        enc_W2.T, enc_b2.reshape(K, 1), proj_W0.T, proj_b0.reshape(K, 1),
        proj_W1.T, proj_b1.reshape(NKP, 1))
    return (xui[0], xuj[0], kpsT.T, bpr[0, 0], kp[0, 0])


# speedup vs baseline: 1.0436x; 1.0010x over previous
import functools

import jax
import jax.numpy as jnp
from jax import lax
from jax.experimental import pallas as pl
from jax.experimental.pallas import tpu as pltpu
from jax.experimental.pallas import tpu_sc as plsc


def _sc_gather3(u, i, j, gamma_u, gamma_i):
    B = u.shape[0]
    K = gamma_u.shape[1]
    info = plsc.get_sparse_core_info()
    nw = info.num_cores * info.num_subcores
    bpw = B // nw
    mesh = plsc.VectorSubcoreMesh(core_axis_name="c", subcore_axis_name="s")

    depth = 96

    @functools.partial(
        pl.kernel,
        out_type=(jax.ShapeDtypeStruct((B, K), jnp.float32),) * 3,
        mesh=mesh,
        scratch_types=[
            pltpu.VMEM((bpw, K), jnp.float32),
            pltpu.VMEM((bpw, K), jnp.float32),
            pltpu.VMEM((bpw, K), jnp.float32),
            pltpu.VMEM((bpw,), jnp.int32),
            pltpu.VMEM((bpw,), jnp.int32),
            pltpu.VMEM((bpw,), jnp.int32),
            pltpu.SemaphoreType.DMA,
        ],
    )
    def body(u_hbm, i_hbm, j_hbm, gu_hbm, gi_hbm, out_u, out_i, out_j,
             rows_u, rows_i, rows_j, su, si, sj, sem):
        wid = lax.axis_index("s") * info.num_cores + lax.axis_index("c")
        base = wid * bpw
        pltpu.sync_copy(u_hbm.at[pl.ds(base, bpw)], su)
        pltpu.sync_copy(i_hbm.at[pl.ds(base, bpw)], si)
        pltpu.sync_copy(j_hbm.at[pl.ds(base, bpw)], sj)

        def _wait3():
            for dst in (rows_u, rows_i, rows_j):
                pltpu.make_async_copy(gu_hbm.at[pl.ds(0, 1)],
                                      dst.at[pl.ds(0, 1)], sem).wait()

        nlane = info.num_lanes

        def issue(g, c):
            vu = su[pl.ds(g * nlane, nlane)]
            vi = si[pl.ds(g * nlane, nlane)]
            vj = sj[pl.ds(g * nlane, nlane)]
            for l in range(nlane):
                r = g * nlane + l
                pltpu.async_copy(gu_hbm.at[pl.ds(vu[l], 1)], rows_u.at[pl.ds(r, 1)], sem)
                pltpu.async_copy(gi_hbm.at[pl.ds(vi[l], 1)], rows_i.at[pl.ds(r, 1)], sem)
                pltpu.async_copy(gi_hbm.at[pl.ds(vj[l], 1)], rows_j.at[pl.ds(r, 1)], sem)

            @pl.when(g >= depth // nlane)
            def _():
                for _ in range(nlane):
                    _wait3()

            return c

        lax.fori_loop(0, bpw // nlane, issue, 0)

        def drain(g, c):
            for _ in range(nlane):
                _wait3()
            return c

        lax.fori_loop(0, min(depth, bpw) // nlane, drain, 0)
        pltpu.sync_copy(rows_u, out_u.at[pl.ds(base, bpw)])
        pltpu.sync_copy(rows_i, out_i.at[pl.ds(base, bpw)])
        pltpu.sync_copy(rows_j, out_j.at[pl.ds(base, bpw)])

    return body(u, i, j, gamma_u, gamma_i)


def _tc_fused(ukT, tkT, gu, gi, gj,
              W0T, b0, W1T, b1, W2T, b2,
              pW0T, pb0, pW1T, pb1, block_n=1024):
    NKP, B = ukT.shape
    K = gu.shape[1]
    nb = B // block_n
    f32 = jnp.float32

    def body(ukT_r, tkT_r, gu_r, gi_r, gj_r, W0T_r, b0_r, W1T_r, b1_r,
             W2T_r, b2_r, pW0T_r, pb0_r, pW1T_r, pb1_r,
             xui_r, xuj_r, kpsT_r, bpr_r, kp_r):
        hT = jnp.dot(W0T_r[...], ukT_r[...], preferred_element_type=f32) + b0_r[...]
        hT = jnp.maximum(jnp.dot(W1T_r[...], hT, preferred_element_type=f32) + b1_r[...], 0.0)
        hT = jnp.maximum(jnp.dot(W2T_r[...], hT, preferred_element_type=f32) + b2_r[...], 0.0)
        guT = gu_r[...].T
        giT = gi_r[...].T
        gjT = gj_r[...].T
        luT = hT + guT
        xui = jnp.sum(luT * giT, axis=0, keepdims=True)
        xuj = jnp.sum(luT * gjT, axis=0, keepdims=True)
        xui_r[...] = xui
        xuj_r[...] = xuj
        phT = jnp.maximum(
            jnp.dot(pW0T_r[...], luT + giT, preferred_element_type=f32) + pb0_r[...], 0.0)
        logitsT = jnp.dot(pW1T_r[...], phT, preferred_element_type=f32) + pb1_r[...]
        kpsT_r[...] = logitsT
        t = tkT_r[...]
        bce = (jnp.maximum(logitsT, 0.0) - logitsT * t
               + jnp.log1p(jnp.exp(-jnp.abs(logitsT))))
        z = xui - xuj
        logsig = jnp.minimum(z, 0.0) - jnp.log1p(jnp.exp(-jnp.abs(z)))

        @pl.when(pl.program_id(0) == 0)
        def _init():
            bpr_r[...] = jnp.zeros((1, 1), f32)
            kp_r[...] = jnp.zeros((1, 1), f32)

        bpr_r[...] += -jnp.sum(logsig)
        kp_r[...] += jnp.sum(bce)

        @pl.when(pl.program_id(0) == nb - 1)
        def _finish():
            bpr_r[...] = bpr_r[...] * (1.0 / B)
            kp_r[...] = kp_r[...] * (1.0 / (B * NKP))

    col = lambda b: (0, b)
    row = lambda b: (b, 0)
    rep = lambda b: (0, 0)
    in_specs = [
        pl.BlockSpec((NKP, block_n), col),
        pl.BlockSpec((NKP, block_n), col),
        pl.BlockSpec((block_n, K), row),
        pl.BlockSpec((block_n, K), row),
        pl.BlockSpec((block_n, K), row),
        pl.BlockSpec((K, NKP), rep),
        pl.BlockSpec((K, 1), rep),
        pl.BlockSpec((K, K), rep),
        pl.BlockSpec((K, 1), rep),
        pl.BlockSpec((K, K), rep),
        pl.BlockSpec((K, 1), rep),
        pl.BlockSpec((K, K), rep),
        pl.BlockSpec((K, 1), rep),
        pl.BlockSpec((NKP, K), rep),
        pl.BlockSpec((NKP, 1), rep),
    ]
    out_specs = [
        pl.BlockSpec((1, block_n), col),
        pl.BlockSpec((1, block_n), col),
        pl.BlockSpec((NKP, block_n), col),
        pl.BlockSpec((1, 1), rep),
        pl.BlockSpec((1, 1), rep),
    ]
    out_shape = [
        jax.ShapeDtypeStruct((1, B), f32),
        jax.ShapeDtypeStruct((1, B), f32),
        jax.ShapeDtypeStruct((NKP, B), f32),
        jax.ShapeDtypeStruct((1, 1), f32),
        jax.ShapeDtypeStruct((1, 1), f32),
    ]
    return pl.pallas_call(
        body,
        grid=(nb,),
        in_specs=in_specs,
        out_specs=out_specs,
        out_shape=out_shape,
    )(ukT, tkT, gu, gi, gj,
      W0T, b0, W1T, b1, W2T, b2, pW0T, pb0, pW1T, pb1)


def kernel(u, i, j, target_kps, user_kps, gamma_i, gamma_u,
           enc_W0, enc_b0, enc_W1, enc_b1, enc_W2, enc_b2,
           proj_W0, proj_b0, proj_W1, proj_b1):
    K = gamma_u.shape[1]
    NKP = user_kps.shape[1]
    gu, gi, gj = _sc_gather3(u.astype(jnp.int32), i.astype(jnp.int32),
                             j.astype(jnp.int32), gamma_u, gamma_i)
    xui, xuj, kpsT, bpr, kp = _tc_fused(
        user_kps.T, target_kps.T, gu, gi, gj,
        enc_W0.T, enc_b0.reshape(K, 1), enc_W1.T, enc_b1.reshape(K, 1),
        enc_W2.T, enc_b2.reshape(K, 1), proj_W0.T, proj_b0.reshape(K, 1),
        proj_W1.T, proj_b1.reshape(NKP, 1))
    return (xui[0], xuj[0], kpsT.T, bpr[0, 0], kp[0, 0])
